# Initial kernel scaffold; baseline (speedup 1.0000x reference)
import jax
import jax.numpy as jnp
from jax.experimental import pallas as pl


def kernel(inputs, graph_mask, bias):
    raise NotImplementedError("write your pallas kernel here")


# speedup vs baseline: 1.5477x; 1.5477x over previous
import functools

import jax
import jax.numpy as jnp
from jax import lax
from jax.experimental import pallas as pl
from jax.experimental.pallas import tpu as pltpu
from jax.experimental.pallas import tpu_sc as plsc

N_NODES = 50000
D_FEAT = 512
NUM_SEGMENTS = 1024
EPSILON = 1e-06

_L = 16
_DV = D_FEAT // _L
_NW = 32

_C1 = 400
_NCH1 = N_NODES // _C1
_UNROLL = 8

_C3 = 80
_NCH3 = N_NODES // _C3
_MAXI3 = -(-_NCH3 // _NW)

_mesh = plsc.VectorSubcoreMesh(core_axis_name="c", subcore_axis_name="s")


@functools.partial(
    pl.kernel,
    out_type=(
        jax.ShapeDtypeStruct((NUM_SEGMENTS, D_FEAT), jnp.float32),
        jax.ShapeDtypeStruct((NUM_SEGMENTS, D_FEAT), jnp.float32),
        jax.ShapeDtypeStruct((_NW * NUM_SEGMENTS, _L), jnp.float32),
    ),
    mesh=_mesh,
    scratch_types=[
        pltpu.VMEM((N_NODES,), jnp.int32),
        pltpu.VMEM((_C1, _L), jnp.float32),
        pltpu.VMEM((_C1, _L), jnp.float32),
        pltpu.VMEM((NUM_SEGMENTS, _L), jnp.float32),
        pltpu.VMEM((NUM_SEGMENTS, _L), jnp.float32),
        pltpu.VMEM((NUM_SEGMENTS, _L), jnp.float32),
        pltpu.SemaphoreType.DMA,
        pltpu.SemaphoreType.DMA,
    ],
    compiler_params=pltpu.CompilerParams(use_tc_tiling_on_sc=False, needs_layout_passes=False),
)
def _k_reduce(x_hbm, ids_hbm, s1_out, s2_out, cnt_out,
              iv, xb0, xb1, acc1, acc2, accc, sem0, sem1):
    cid = lax.axis_index("c")
    sid = lax.axis_index("s")
    w = sid * 2 + cid
    f0 = w * _L

    zeros16 = jnp.zeros((_L,), jnp.float32)
    ones16 = jnp.ones((_L,), jnp.float32)
    cols = lax.iota(jnp.int32, _L)

    idcp = pltpu.async_copy(ids_hbm, iv, sem0)

    def _init(r, carry):
        acc1[r, :] = zeros16
        acc2[r, :] = zeros16
        accc[r, :] = zeros16
        return carry
    lax.fori_loop(0, NUM_SEGMENTS, _init, 0)
    idcp.wait()

    bufs = (xb0, xb1)
    sems = (sem0, sem1)

    def _issue(k, b):
        pltpu.async_copy(
            x_hbm.at[pl.ds(k * _C1, _C1), pl.ds(f0, _L)], bufs[b], sems[b])

    def _wait(b):
        pltpu.make_async_copy(
            x_hbm.at[pl.ds(0, _C1), pl.ds(f0, _L)], bufs[b], sems[b]).wait()

    def _process(k, b):
        xb = bufs[b]
        rbase = k * _C1

        def _rows(rb, carry):
            base = rb * _UNROLL
            for u in range(_UNROLL):
                r = base + u
                seg = plsc.load_gather(
                    iv, [jnp.full((_L,), rbase + r, jnp.int32)])
                v = xb[r, :]
                plsc.addupdate_scatter(acc1, [seg, cols], v)
                plsc.addupdate_scatter(acc2, [seg, cols], v * v)
            return carry
        lax.fori_loop(0, _C1 // _UNROLL, _rows, 0)

        @pl.when(lax.rem(k, _NW) == w)
        def _():
            def _crows(rb, carry):
                base = rb * _UNROLL
                for u in range(_UNROLL):
                    r = base + u
                    seg = plsc.load_gather(
                        iv, [jnp.full((_L,), rbase + r, jnp.int32)])
                    plsc.addupdate_scatter(accc, [seg, cols], ones16)
                return carry
            lax.fori_loop(0, _C1 // _UNROLL, _crows, 0)

    _issue(0, 0)
    _issue(1, 1)

    def _outer(ko, carry):
        k = ko * 2
        for b in range(2):
            kk = k + b
            _wait(b)
            _process(kk, b)

            @pl.when(kk + 2 < _NCH1)
            def _():
                _issue(kk + 2, b)
        return carry
    lax.fori_loop(0, (_NCH1 - 1) // 2, _outer, 0)
    _wait((_NCH1 - 1) % 2)
    _process(_NCH1 - 1, (_NCH1 - 1) % 2)

    pltpu.sync_copy(acc1, s1_out.at[pl.ds(0, NUM_SEGMENTS), pl.ds(f0, _L)])
    pltpu.sync_copy(acc2, s2_out.at[pl.ds(0, NUM_SEGMENTS), pl.ds(f0, _L)])
    pltpu.sync_copy(accc, cnt_out.at[pl.ds(w * NUM_SEGMENTS, NUM_SEGMENTS)])


def _k_table_body(s1_ref, s2_ref, cnt_ref, bias_ref, t_ref):
    s1 = s1_ref[...]
    s2 = s2_ref[...]
    cnt = jnp.sum(
        cnt_ref[:, 0].reshape(_NW, NUM_SEGMENTS), axis=0)
    c = jnp.maximum(cnt, 1.0)[:, None]
    mean = s1 / c
    b = bias_ref[0]
    var = s2 / c - mean * mean + b * b
    a = lax.rsqrt(var + EPSILON)
    t_ref[:, :D_FEAT] = a
    t_ref[:, D_FEAT:] = (b - mean) * a


@functools.partial(
    pl.kernel,
    out_type=jax.ShapeDtypeStruct((N_NODES, D_FEAT), jnp.float32),
    mesh=_mesh,
    scratch_types=[
        pltpu.VMEM((_C3, D_FEAT), jnp.float32),
        pltpu.VMEM((_C3, 2 * D_FEAT), jnp.float32),
        pltpu.VMEM((_C3,), jnp.int32),
        pltpu.SemaphoreType.DMA,
    ],
    compiler_params=pltpu.CompilerParams(use_tc_tiling_on_sc=False, needs_layout_passes=False),
)
def _k_apply(x_hbm, ids_hbm, t_hbm, out_hbm, xbuf, tbuf, idx, sem):
    cid = lax.axis_index("c")
    sid = lax.axis_index("s")
    wid = sid * 2 + cid

    def _chunk(i, carry):
        k = wid + i * _NW

        @pl.when(k < _NCH3)
        def _():
            rbase = k * _C3
            pltpu.sync_copy(ids_hbm.at[pl.ds(rbase, _C3)], idx)
            gcp = pltpu.async_copy(t_hbm.at[idx], tbuf, sem)
            pltpu.sync_copy(x_hbm.at[pl.ds(rbase, _C3)], xbuf)
            gcp.wait()

            def _row(r, carry2):
                def _col(c, carry3):
                    a = tbuf[r, pl.ds(c * _L, _L)]
                    b = tbuf[r, pl.ds(D_FEAT + c * _L, _L)]
                    v = xbuf[r, pl.ds(c * _L, _L)]
                    xbuf[r, pl.ds(c * _L, _L)] = v * a + b
                    return carry3
                return lax.fori_loop(0, _DV, _col, carry2)
            lax.fori_loop(0, _C3, _row, 0)

            pltpu.sync_copy(xbuf, out_hbm.at[pl.ds(rbase, _C3)])
        return carry
    lax.fori_loop(0, _MAXI3, _chunk, 0)


def kernel(inputs, graph_mask, bias):
    ids = graph_mask.astype(jnp.int32)
    s1, s2, cnt = _k_reduce(inputs, ids)
    table = pl.pallas_call(
        _k_table_body,
        out_shape=jax.ShapeDtypeStruct((NUM_SEGMENTS, 2 * D_FEAT), jnp.float32),
    )(s1, s2, cnt, bias.reshape(1, D_FEAT))
    return _k_apply(inputs, ids, table)


# speedup vs baseline: 1.8216x; 1.1770x over previous
import functools

import jax
import jax.numpy as jnp
from jax import lax
from jax.experimental import pallas as pl
from jax.experimental.pallas import tpu as pltpu
from jax.experimental.pallas import tpu_sc as plsc

N_NODES = 50000
D_FEAT = 512
NUM_SEGMENTS = 1024
EPSILON = 1e-06

_L = 16
_DV = D_FEAT // _L
_NW = 32

_C1 = 400
_NCH1 = N_NODES // _C1
_NB1 = 4
_UNROLL = 8

_C3 = 40
_NCH3 = N_NODES // _C3
_MAXI3 = -(-_NCH3 // _NW)

_params = pltpu.CompilerParams(use_tc_tiling_on_sc=False,
                               needs_layout_passes=False)
_mesh = plsc.VectorSubcoreMesh(core_axis_name="c", subcore_axis_name="s")


@functools.partial(
    pl.kernel,
    out_type=(
        jax.ShapeDtypeStruct((NUM_SEGMENTS, D_FEAT), jnp.float32),
        jax.ShapeDtypeStruct((NUM_SEGMENTS, D_FEAT), jnp.float32),
        jax.ShapeDtypeStruct((_NW * NUM_SEGMENTS, _L), jnp.float32),
    ),
    mesh=_mesh,
    scratch_types=[
        pltpu.VMEM((N_NODES,), jnp.int32),
        pltpu.VMEM((_C1, _L), jnp.float32),
        pltpu.VMEM((_C1, _L), jnp.float32),
        pltpu.VMEM((_C1, _L), jnp.float32),
        pltpu.VMEM((_C1, _L), jnp.float32),
        pltpu.VMEM((NUM_SEGMENTS, _L), jnp.float32),
        pltpu.VMEM((NUM_SEGMENTS, _L), jnp.float32),
        pltpu.VMEM((NUM_SEGMENTS, _L), jnp.float32),
        pltpu.SemaphoreType.DMA,
        pltpu.SemaphoreType.DMA,
        pltpu.SemaphoreType.DMA,
        pltpu.SemaphoreType.DMA,
    ],
    compiler_params=_params,
)
def _k_reduce(x_hbm, ids_hbm, s1_out, s2_out, cnt_out,
              iv, xb0, xb1, xb2, xb3, acc1, acc2, accc,
              sem0, sem1, sem2, sem3):
    cid = lax.axis_index("c")
    sid = lax.axis_index("s")
    w = sid * 2 + cid
    f0 = w * _L

    zeros16 = jnp.zeros((_L,), jnp.float32)
    ones16 = jnp.ones((_L,), jnp.float32)
    cols = lax.iota(jnp.int32, _L)

    idcp = pltpu.async_copy(ids_hbm, iv, sem0)

    def _init(r, carry):
        acc1[r, :] = zeros16
        acc2[r, :] = zeros16
        accc[r, :] = zeros16
        return carry
    lax.fori_loop(0, NUM_SEGMENTS, _init, 0)
    idcp.wait()

    bufs = (xb0, xb1, xb2, xb3)
    sems = (sem0, sem1, sem2, sem3)

    def _issue(k, b):
        pltpu.async_copy(
            x_hbm.at[pl.ds(k * _C1, _C1), pl.ds(f0, _L)], bufs[b], sems[b])

    def _wait(b):
        pltpu.make_async_copy(
            x_hbm.at[pl.ds(0, _C1), pl.ds(f0, _L)], bufs[b], sems[b]).wait()

    def _process(k, b):
        xb = bufs[b]
        rbase = k * _C1

        def _rows(rb, carry):
            base = rb * _UNROLL
            for u in range(_UNROLL):
                r = base + u
                seg = plsc.load_gather(
                    iv, [jnp.full((_L,), rbase + r, jnp.int32)])
                v = xb[r, :]
                plsc.addupdate_scatter(acc1, [seg, cols], v)
                plsc.addupdate_scatter(acc2, [seg, cols], v * v)
            return carry
        lax.fori_loop(0, _C1 // _UNROLL, _rows, 0)

        @pl.when(lax.rem(k, _NW) == w)
        def _():
            def _crows(rb, carry):
                base = rb * _UNROLL
                for u in range(_UNROLL):
                    r = base + u
                    seg = plsc.load_gather(
                        iv, [jnp.full((_L,), rbase + r, jnp.int32)])
                    plsc.addupdate_scatter(accc, [seg, cols], ones16)
                return carry
            lax.fori_loop(0, _C1 // _UNROLL, _crows, 0)

    for b in range(_NB1):
        _issue(b, b)

    def _outer(ko, carry):
        k = ko * _NB1
        for b in range(_NB1):
            kk = k + b
            _wait(b)
            _process(kk, b)

            @pl.when(kk + _NB1 < _NCH1)
            def _():
                _issue(kk + _NB1, b)
        return carry
    lax.fori_loop(0, _NCH1 // _NB1, _outer, 0)
    _wait(0)
    _process(_NCH1 - 1, 0)

    pltpu.sync_copy(acc1, s1_out.at[pl.ds(0, NUM_SEGMENTS), pl.ds(f0, _L)])
    pltpu.sync_copy(acc2, s2_out.at[pl.ds(0, NUM_SEGMENTS), pl.ds(f0, _L)])
    pltpu.sync_copy(accc, cnt_out.at[pl.ds(w * NUM_SEGMENTS, NUM_SEGMENTS)])


def _k_table_body(s1_ref, s2_ref, cnt_ref, bias_ref, t_ref):
    s1 = s1_ref[...]
    s2 = s2_ref[...]
    cnt = jnp.sum(
        cnt_ref[:, 0].reshape(_NW, NUM_SEGMENTS), axis=0)
    c = jnp.maximum(cnt, 1.0)[:, None]
    mean = s1 / c
    b = bias_ref[0]
    var = s2 / c - mean * mean + b * b
    a = lax.rsqrt(var + EPSILON)
    t_ref[:, :D_FEAT] = a
    t_ref[:, D_FEAT:] = (b - mean) * a


@functools.partial(
    pl.kernel,
    out_type=jax.ShapeDtypeStruct((N_NODES, D_FEAT), jnp.float32),
    mesh=_mesh,
    scratch_types=[
        pltpu.VMEM((_C3, D_FEAT), jnp.float32),
        pltpu.VMEM((_C3, D_FEAT), jnp.float32),
        pltpu.VMEM((_C3, 2 * D_FEAT), jnp.float32),
        pltpu.VMEM((_C3, 2 * D_FEAT), jnp.float32),
        pltpu.VMEM((_MAXI3 * _C3,), jnp.int32),
        pltpu.SemaphoreType.DMA,
        pltpu.SemaphoreType.DMA,
        pltpu.SemaphoreType.DMA,
        pltpu.SemaphoreType.DMA,
        pltpu.SemaphoreType.DMA,
    ],
    compiler_params=_params,
)
def _k_apply(x_hbm, ids_hbm, t_hbm, out_hbm,
             xb0, xb1, tb0, tb1, ivall,
             gsem0, gsem1, osem0, osem1, isem):
    cid = lax.axis_index("c")
    sid = lax.axis_index("s")
    wid = sid * 2 + cid

    xbufs = (xb0, xb1)
    tbufs = (tb0, tb1)
    gsems = (gsem0, gsem1)
    osems = (osem0, osem1)

    for j in range(_MAXI3):
        kj = wid + j * _NW

        @pl.when(kj < _NCH3)
        def _():
            pltpu.async_copy(ids_hbm.at[pl.ds(kj * _C3, _C3)],
                             ivall.at[pl.ds(j * _C3, _C3)], isem)
    for j in range(_MAXI3):
        kj = wid + j * _NW

        @pl.when(kj < _NCH3)
        def _():
            pltpu.make_async_copy(ids_hbm.at[pl.ds(0, _C3)],
                                  ivall.at[pl.ds(0, _C3)], isem).wait()

    def _issue_pre_b(i, b):
        k = wid + i * _NW
        pltpu.async_copy(t_hbm.at[ivall.at[pl.ds(i * _C3, _C3)]],
                         tbufs[b], gsems[b])
        pltpu.async_copy(x_hbm.at[pl.ds(k * _C3, _C3)], xbufs[b], gsems[b])

    def _wait_pre(b):
        pltpu.make_async_copy(t_hbm.at[pl.ds(0, _C3)],
                              tbufs[b], gsems[b]).wait()
        pltpu.make_async_copy(x_hbm.at[pl.ds(0, _C3)],
                              xbufs[b], gsems[b]).wait()

    def _wait_out(b):
        pltpu.make_async_copy(
            tbufs[b].at[pl.ds(0, _C3), pl.ds(0, D_FEAT)],
            out_hbm.at[pl.ds(0, _C3)], osems[b]).wait()

    _issue_pre_b(0, 0)

    def _outer(io, carry):
        i0 = io * 2
        for b in range(2):
            i = i0 + b
            k = wid + i * _NW

            @pl.when(k < _NCH3)
            def _():
                @pl.when(i >= 1)
                def _():
                    _wait_out(1 - b)

                @pl.when(k + _NW < _NCH3)
                def _():
                    _issue_pre_b(i + 1, 1 - b)

                _wait_pre(b)

                xb = xbufs[b]
                tb = tbufs[b]

                def _row(r, carry2):
                    def _col(c, carry3):
                        a = tb[r, pl.ds(c * _L, _L)]
                        bv = tb[r, pl.ds(D_FEAT + c * _L, _L)]
                        v = xb[r, pl.ds(c * _L, _L)]
                        tb[r, pl.ds(c * _L, _L)] = v * a + bv
                        return carry3
                    return lax.fori_loop(0, _DV, _col, carry2)
                lax.fori_loop(0, _C3, _row, 0)

                pltpu.async_copy(
                    tb.at[pl.ds(0, _C3), pl.ds(0, D_FEAT)],
                    out_hbm.at[pl.ds(k * _C3, _C3)], osems[b])
        return carry
    lax.fori_loop(0, _MAXI3 // 2, _outer, 0)

    @pl.when(wid < _NCH3 - (_MAXI3 - 1) * _NW)
    def _():
        _wait_out(1)

    @pl.when(wid >= _NCH3 - (_MAXI3 - 1) * _NW)
    def _():
        _wait_out(0)


def kernel(inputs, graph_mask, bias):
    ids = graph_mask.astype(jnp.int32)
    s1, s2, cnt = _k_reduce(inputs, ids)
    table = pl.pallas_call(
        _k_table_body,
        out_shape=jax.ShapeDtypeStruct((NUM_SEGMENTS, 2 * D_FEAT), jnp.float32),
    )(s1, s2, cnt, bias.reshape(1, D_FEAT))
    return _k_apply(inputs, ids, table)


# speedup vs baseline: 1.8472x; 1.0140x over previous
import functools

import jax
import jax.numpy as jnp
from jax import lax
from jax.experimental import pallas as pl
from jax.experimental.pallas import tpu as pltpu
from jax.experimental.pallas import tpu_sc as plsc

N_NODES = 50000
D_FEAT = 512
NUM_SEGMENTS = 1024
EPSILON = 1e-06

_L = 16
_DV = D_FEAT // _L
_NW = 32

_C1 = 400
_NCH1 = N_NODES // _C1
_NB1 = 4
_UNROLL = 8
_STRIDE = _C1 // _UNROLL

_C3 = 40
_NCH3 = N_NODES // _C3
_MAXI3 = -(-_NCH3 // _NW)

_params = pltpu.CompilerParams(use_tc_tiling_on_sc=False,
                               needs_layout_passes=False)
_mesh = plsc.VectorSubcoreMesh(core_axis_name="c", subcore_axis_name="s")


@functools.partial(
    pl.kernel,
    out_type=(
        jax.ShapeDtypeStruct((NUM_SEGMENTS, D_FEAT), jnp.float32),
        jax.ShapeDtypeStruct((NUM_SEGMENTS, D_FEAT), jnp.float32),
        jax.ShapeDtypeStruct((_NW * NUM_SEGMENTS, _L), jnp.float32),
    ),
    mesh=_mesh,
    scratch_types=[
        pltpu.VMEM((N_NODES,), jnp.int32),
        pltpu.VMEM((_C1, _L), jnp.float32),
        pltpu.VMEM((_C1, _L), jnp.float32),
        pltpu.VMEM((_C1, _L), jnp.float32),
        pltpu.VMEM((_C1, _L), jnp.float32),
        pltpu.VMEM((NUM_SEGMENTS, _L), jnp.float32),
        pltpu.VMEM((NUM_SEGMENTS, _L), jnp.float32),
        pltpu.VMEM((NUM_SEGMENTS, _L), jnp.float32),
        pltpu.SemaphoreType.DMA,
        pltpu.SemaphoreType.DMA,
        pltpu.SemaphoreType.DMA,
        pltpu.SemaphoreType.DMA,
    ],
    compiler_params=_params,
)
def _k_reduce(x_hbm, ids_hbm, s1_out, s2_out, cnt_out,
              iv, xb0, xb1, xb2, xb3, acc1, acc2, accc,
              sem0, sem1, sem2, sem3):
    cid = lax.axis_index("c")
    sid = lax.axis_index("s")
    w = sid * 2 + cid
    f0 = w * _L

    zeros16 = jnp.zeros((_L,), jnp.float32)
    ones16 = jnp.ones((_L,), jnp.float32)
    cols = lax.iota(jnp.int32, _L)

    idcp = pltpu.async_copy(ids_hbm, iv, sem0)

    def _init(r, carry):
        acc1[r, :] = zeros16
        acc2[r, :] = zeros16
        accc[r, :] = zeros16
        return carry
    lax.fori_loop(0, NUM_SEGMENTS, _init, 0)
    idcp.wait()

    bufs = (xb0, xb1, xb2, xb3)
    sems = (sem0, sem1, sem2, sem3)

    def _issue(k, b):
        pltpu.async_copy(
            x_hbm.at[pl.ds(k * _C1, _C1), pl.ds(f0, _L)], bufs[b], sems[b])

    def _wait(b):
        pltpu.make_async_copy(
            x_hbm.at[pl.ds(0, _C1), pl.ds(f0, _L)], bufs[b], sems[b]).wait()

    def _process(k, b):
        xb = bufs[b]
        rbase = k * _C1

        def _rows(jj, carry):
            for p in range(_UNROLL):
                r = p * _STRIDE + jj
                seg = plsc.load_gather(
                    iv, [jnp.full((_L,), rbase + r, jnp.int32)])
                v = xb[r, :]
                plsc.addupdate_scatter(acc1, [seg, cols], v)
                plsc.addupdate_scatter(acc2, [seg, cols], v * v)
            return carry
        lax.fori_loop(0, _STRIDE, _rows, 0)

        @pl.when(lax.rem(k, _NW) == w)
        def _():
            def _crows(jj, carry):
                for p in range(_UNROLL):
                    r = p * _STRIDE + jj
                    seg = plsc.load_gather(
                        iv, [jnp.full((_L,), rbase + r, jnp.int32)])
                    plsc.addupdate_scatter(accc, [seg, cols], ones16)
                return carry
            lax.fori_loop(0, _STRIDE, _crows, 0)

    for b in range(_NB1):
        _issue(b, b)

    def _outer(ko, carry):
        k = ko * _NB1
        for b in range(_NB1):
            kk = k + b
            _wait(b)
            _process(kk, b)

            @pl.when(kk + _NB1 < _NCH1)
            def _():
                _issue(kk + _NB1, b)
        return carry
    lax.fori_loop(0, _NCH1 // _NB1, _outer, 0)
    _wait(0)
    _process(_NCH1 - 1, 0)

    pltpu.sync_copy(acc1, s1_out.at[pl.ds(0, NUM_SEGMENTS), pl.ds(f0, _L)])
    pltpu.sync_copy(acc2, s2_out.at[pl.ds(0, NUM_SEGMENTS), pl.ds(f0, _L)])
    pltpu.sync_copy(accc, cnt_out.at[pl.ds(w * NUM_SEGMENTS, NUM_SEGMENTS)])


def _k_table_body(s1_ref, s2_ref, cnt_ref, bias_ref, t_ref):
    s1 = s1_ref[...]
    s2 = s2_ref[...]
    cnt = jnp.sum(
        cnt_ref[:, 0].reshape(_NW, NUM_SEGMENTS), axis=0)
    c = jnp.maximum(cnt, 1.0)[:, None]
    mean = s1 / c
    b = bias_ref[0]
    var = s2 / c - mean * mean + b * b
    a = lax.rsqrt(var + EPSILON)
    t_ref[:, :D_FEAT] = a
    t_ref[:, D_FEAT:] = (b - mean) * a


@functools.partial(
    pl.kernel,
    out_type=jax.ShapeDtypeStruct((N_NODES, D_FEAT), jnp.float32),
    mesh=_mesh,
    scratch_types=[
        pltpu.VMEM((_C3, D_FEAT), jnp.float32),
        pltpu.VMEM((_C3, D_FEAT), jnp.float32),
        pltpu.VMEM((_C3, 2 * D_FEAT), jnp.float32),
        pltpu.VMEM((_C3, 2 * D_FEAT), jnp.float32),
        pltpu.VMEM((_MAXI3 * _C3,), jnp.int32),
        pltpu.SemaphoreType.DMA,
        pltpu.SemaphoreType.DMA,
        pltpu.SemaphoreType.DMA,
        pltpu.SemaphoreType.DMA,
        pltpu.SemaphoreType.DMA,
    ],
    compiler_params=_params,
)
def _k_apply(x_hbm, ids_hbm, t_hbm, out_hbm,
             xb0, xb1, tb0, tb1, ivall,
             gsem0, gsem1, osem0, osem1, isem):
    cid = lax.axis_index("c")
    sid = lax.axis_index("s")
    wid = sid * 2 + cid

    xbufs = (xb0, xb1)
    tbufs = (tb0, tb1)
    gsems = (gsem0, gsem1)
    osems = (osem0, osem1)

    for j in range(_MAXI3):
        kj = wid + j * _NW

        @pl.when(kj < _NCH3)
        def _():
            pltpu.async_copy(ids_hbm.at[pl.ds(kj * _C3, _C3)],
                             ivall.at[pl.ds(j * _C3, _C3)], isem)
    for j in range(_MAXI3):
        kj = wid + j * _NW

        @pl.when(kj < _NCH3)
        def _():
            pltpu.make_async_copy(ids_hbm.at[pl.ds(0, _C3)],
                                  ivall.at[pl.ds(0, _C3)], isem).wait()

    def _issue_pre_b(i, b):
        k = wid + i * _NW
        pltpu.async_copy(t_hbm.at[ivall.at[pl.ds(i * _C3, _C3)]],
                         tbufs[b], gsems[b])
        pltpu.async_copy(x_hbm.at[pl.ds(k * _C3, _C3)], xbufs[b], gsems[b])

    def _wait_pre(b):
        pltpu.make_async_copy(t_hbm.at[pl.ds(0, _C3)],
                              tbufs[b], gsems[b]).wait()
        pltpu.make_async_copy(x_hbm.at[pl.ds(0, _C3)],
                              xbufs[b], gsems[b]).wait()

    def _wait_out(b):
        pltpu.make_async_copy(
            tbufs[b].at[pl.ds(0, _C3), pl.ds(0, D_FEAT)],
            out_hbm.at[pl.ds(0, _C3)], osems[b]).wait()

    _issue_pre_b(0, 0)

    def _outer(io, carry):
        i0 = io * 2
        for b in range(2):
            i = i0 + b
            k = wid + i * _NW

            @pl.when(k < _NCH3)
            def _():
                @pl.when(i >= 1)
                def _():
                    _wait_out(1 - b)

                @pl.when(k + _NW < _NCH3)
                def _():
                    _issue_pre_b(i + 1, 1 - b)

                _wait_pre(b)

                xb = xbufs[b]
                tb = tbufs[b]

                def _row(r, carry2):
                    def _col(c, carry3):
                        a = tb[r, pl.ds(c * _L, _L)]
                        bv = tb[r, pl.ds(D_FEAT + c * _L, _L)]
                        v = xb[r, pl.ds(c * _L, _L)]
                        tb[r, pl.ds(c * _L, _L)] = v * a + bv
                        return carry3
                    return lax.fori_loop(0, _DV, _col, carry2)
                lax.fori_loop(0, _C3, _row, 0)

                pltpu.async_copy(
                    tb.at[pl.ds(0, _C3), pl.ds(0, D_FEAT)],
                    out_hbm.at[pl.ds(k * _C3, _C3)], osems[b])
        return carry
    lax.fori_loop(0, _MAXI3 // 2, _outer, 0)

    @pl.when(wid < _NCH3 - (_MAXI3 - 1) * _NW)
    def _():
        _wait_out(1)

    @pl.when(wid >= _NCH3 - (_MAXI3 - 1) * _NW)
    def _():
        _wait_out(0)


def kernel(inputs, graph_mask, bias):
    ids = graph_mask.astype(jnp.int32)
    s1, s2, cnt = _k_reduce(inputs, ids)
    table = pl.pallas_call(
        _k_table_body,
        out_shape=jax.ShapeDtypeStruct((NUM_SEGMENTS, 2 * D_FEAT), jnp.float32),
    )(s1, s2, cnt, bias.reshape(1, D_FEAT))
    return _k_apply(inputs, ids, table)


# speedup vs baseline: 2.1583x; 1.1684x over previous
import functools

import jax
import jax.numpy as jnp
from jax import lax
from jax.experimental import pallas as pl
from jax.experimental.pallas import tpu as pltpu
from jax.experimental.pallas import tpu_sc as plsc

N_NODES = 50000
D_FEAT = 512
NUM_SEGMENTS = 1024
EPSILON = 1e-06

_L = 16
_DV = D_FEAT // _L
_NW = 32

_NH = 2
_NG = 16
_GW = 2 * _L
_HROWS = N_NODES // _NH
_C1 = 250
_NCH1 = _HROWS // _C1
_UNROLL = 5
_STRIDE = _C1 // _UNROLL

_C3 = 40
_NCH3 = N_NODES // _C3
_MAXI3 = -(-_NCH3 // _NW)

_params = pltpu.CompilerParams(use_tc_tiling_on_sc=False,
                               needs_layout_passes=False)
_mesh = plsc.VectorSubcoreMesh(core_axis_name="c", subcore_axis_name="s")


@functools.partial(
    pl.kernel,
    out_type=(
        jax.ShapeDtypeStruct((_NH * NUM_SEGMENTS, D_FEAT), jnp.float32),
        jax.ShapeDtypeStruct((_NH * NUM_SEGMENTS, D_FEAT), jnp.float32),
        jax.ShapeDtypeStruct((_NW * NUM_SEGMENTS, _L), jnp.float32),
    ),
    mesh=_mesh,
    scratch_types=[
        pltpu.VMEM((_HROWS,), jnp.int32),
        pltpu.VMEM((_C1, _GW), jnp.float32),
        pltpu.VMEM((_C1, _GW), jnp.float32),
        pltpu.VMEM((NUM_SEGMENTS, _GW), jnp.float32),
        pltpu.VMEM((NUM_SEGMENTS, _GW), jnp.float32),
        pltpu.VMEM((NUM_SEGMENTS, _L), jnp.float32),
        pltpu.SemaphoreType.DMA,
        pltpu.SemaphoreType.DMA,
    ],
    compiler_params=_params,
)
def _k_reduce(x_hbm, ids_hbm, s1_out, s2_out, cnt_out,
              iv, xb0, xb1, acc1, acc2, accc, sem0, sem1):
    cid = lax.axis_index("c")
    sid = lax.axis_index("s")
    w = sid * 2 + cid
    h = w // _NG
    g = lax.rem(w, _NG)
    f0 = g * _GW
    row0 = h * _HROWS

    zeros16 = jnp.zeros((_L,), jnp.float32)
    ones16 = jnp.ones((_L,), jnp.float32)
    cols = lax.iota(jnp.int32, _L)
    cols2 = cols + _L

    idcp = pltpu.async_copy(ids_hbm.at[pl.ds(row0, _HROWS)], iv, sem0)

    def _init(r, carry):
        acc1[r, pl.ds(0, _L)] = zeros16
        acc1[r, pl.ds(_L, _L)] = zeros16
        acc2[r, pl.ds(0, _L)] = zeros16
        acc2[r, pl.ds(_L, _L)] = zeros16
        accc[r, :] = zeros16
        return carry
    lax.fori_loop(0, NUM_SEGMENTS, _init, 0)
    idcp.wait()

    bufs = (xb0, xb1)
    sems = (sem0, sem1)

    def _issue(k, b):
        pltpu.async_copy(
            x_hbm.at[pl.ds(row0 + k * _C1, _C1), pl.ds(f0, _GW)],
            bufs[b], sems[b])

    def _wait(b):
        pltpu.make_async_copy(
            x_hbm.at[pl.ds(0, _C1), pl.ds(0, _GW)], bufs[b], sems[b]).wait()

    def _process(k, b):
        xb = bufs[b]
        rbase = k * _C1

        def _rows(jj, carry):
            for p in range(_UNROLL):
                r = p * _STRIDE + jj
                seg = plsc.load_gather(
                    iv, [jnp.full((_L,), rbase + r, jnp.int32)])
                v0 = xb[r, pl.ds(0, _L)]
                v1 = xb[r, pl.ds(_L, _L)]
                plsc.addupdate_scatter(acc1, [seg, cols], v0)
                plsc.addupdate_scatter(acc1, [seg, cols2], v1)
                plsc.addupdate_scatter(acc2, [seg, cols], v0 * v0)
                plsc.addupdate_scatter(acc2, [seg, cols2], v1 * v1)
            return carry
        lax.fori_loop(0, _STRIDE, _rows, 0)

        @pl.when(lax.rem(k, _NG) == g)
        def _():
            def _crows(jj, carry):
                for p in range(_UNROLL):
                    r = p * _STRIDE + jj
                    seg = plsc.load_gather(
                        iv, [jnp.full((_L,), rbase + r, jnp.int32)])
                    plsc.addupdate_scatter(accc, [seg, cols], ones16)
                return carry
            lax.fori_loop(0, _STRIDE, _crows, 0)

    _issue(0, 0)
    _issue(1, 1)

    def _outer(ko, carry):
        k = ko * 2
        for b in range(2):
            kk = k + b
            _wait(b)
            _process(kk, b)

            @pl.when(kk + 2 < _NCH1)
            def _():
                _issue(kk + 2, b)
        return carry
    lax.fori_loop(0, _NCH1 // 2, _outer, 0)

    obase = h * NUM_SEGMENTS
    pltpu.sync_copy(
        acc1, s1_out.at[pl.ds(obase, NUM_SEGMENTS), pl.ds(f0, _GW)])
    pltpu.sync_copy(
        acc2, s2_out.at[pl.ds(obase, NUM_SEGMENTS), pl.ds(f0, _GW)])
    pltpu.sync_copy(accc, cnt_out.at[pl.ds(w * NUM_SEGMENTS, NUM_SEGMENTS)])


def _k_table_body(s1_ref, s2_ref, cnt_ref, bias_ref, t_ref):
    s1 = s1_ref[: NUM_SEGMENTS, :] + s1_ref[NUM_SEGMENTS:, :]
    s2 = s2_ref[: NUM_SEGMENTS, :] + s2_ref[NUM_SEGMENTS:, :]
    cnt = jnp.sum(
        cnt_ref[:, 0].reshape(_NW, NUM_SEGMENTS), axis=0)
    c = jnp.maximum(cnt, 1.0)[:, None]
    mean = s1 / c
    b = bias_ref[0]
    var = s2 / c - mean * mean + b * b
    a = lax.rsqrt(var + EPSILON)
    t_ref[:, :D_FEAT] = a
    t_ref[:, D_FEAT:] = (b - mean) * a


@functools.partial(
    pl.kernel,
    out_type=jax.ShapeDtypeStruct((N_NODES, D_FEAT), jnp.float32),
    mesh=_mesh,
    scratch_types=[
        pltpu.VMEM((_C3, D_FEAT), jnp.float32),
        pltpu.VMEM((_C3, D_FEAT), jnp.float32),
        pltpu.VMEM((_C3, 2 * D_FEAT), jnp.float32),
        pltpu.VMEM((_C3, 2 * D_FEAT), jnp.float32),
        pltpu.VMEM((_MAXI3 * _C3,), jnp.int32),
        pltpu.SemaphoreType.DMA,
        pltpu.SemaphoreType.DMA,
        pltpu.SemaphoreType.DMA,
        pltpu.SemaphoreType.DMA,
        pltpu.SemaphoreType.DMA,
    ],
    compiler_params=_params,
)
def _k_apply(x_hbm, ids_hbm, t_hbm, out_hbm,
             xb0, xb1, tb0, tb1, ivall,
             gsem0, gsem1, osem0, osem1, isem):
    cid = lax.axis_index("c")
    sid = lax.axis_index("s")
    wid = sid * 2 + cid

    xbufs = (xb0, xb1)
    tbufs = (tb0, tb1)
    gsems = (gsem0, gsem1)
    osems = (osem0, osem1)

    for j in range(_MAXI3):
        kj = wid + j * _NW

        @pl.when(kj < _NCH3)
        def _():
            pltpu.async_copy(ids_hbm.at[pl.ds(kj * _C3, _C3)],
                             ivall.at[pl.ds(j * _C3, _C3)], isem)
    for j in range(_MAXI3):
        kj = wid + j * _NW

        @pl.when(kj < _NCH3)
        def _():
            pltpu.make_async_copy(ids_hbm.at[pl.ds(0, _C3)],
                                  ivall.at[pl.ds(0, _C3)], isem).wait()

    def _issue_pre_b(i, b):
        k = wid + i * _NW
        pltpu.async_copy(t_hbm.at[ivall.at[pl.ds(i * _C3, _C3)]],
                         tbufs[b], gsems[b])
        pltpu.async_copy(x_hbm.at[pl.ds(k * _C3, _C3)], xbufs[b], gsems[b])

    def _wait_pre(b):
        pltpu.make_async_copy(t_hbm.at[pl.ds(0, _C3)],
                              tbufs[b], gsems[b]).wait()
        pltpu.make_async_copy(x_hbm.at[pl.ds(0, _C3)],
                              xbufs[b], gsems[b]).wait()

    def _wait_out(b):
        pltpu.make_async_copy(
            tbufs[b].at[pl.ds(0, _C3), pl.ds(0, D_FEAT)],
            out_hbm.at[pl.ds(0, _C3)], osems[b]).wait()

    _issue_pre_b(0, 0)

    def _outer(io, carry):
        i0 = io * 2
        for b in range(2):
            i = i0 + b
            k = wid + i * _NW

            @pl.when(k < _NCH3)
            def _():
                @pl.when(i >= 1)
                def _():
                    _wait_out(1 - b)

                @pl.when(k + _NW < _NCH3)
                def _():
                    _issue_pre_b(i + 1, 1 - b)

                _wait_pre(b)

                xb = xbufs[b]
                tb = tbufs[b]

                def _row(r, carry2):
                    def _col(c, carry3):
                        a = tb[r, pl.ds(c * _L, _L)]
                        bv = tb[r, pl.ds(D_FEAT + c * _L, _L)]
                        v = xb[r, pl.ds(c * _L, _L)]
                        tb[r, pl.ds(c * _L, _L)] = v * a + bv
                        return carry3
                    return lax.fori_loop(0, _DV, _col, carry2)
                lax.fori_loop(0, _C3, _row, 0)

                pltpu.async_copy(
                    tb.at[pl.ds(0, _C3), pl.ds(0, D_FEAT)],
                    out_hbm.at[pl.ds(k * _C3, _C3)], osems[b])
        return carry
    lax.fori_loop(0, _MAXI3 // 2, _outer, 0)

    @pl.when(wid < _NCH3 - (_MAXI3 - 1) * _NW)
    def _():
        _wait_out(1)

    @pl.when(wid >= _NCH3 - (_MAXI3 - 1) * _NW)
    def _():
        _wait_out(0)


def kernel(inputs, graph_mask, bias):
    ids = graph_mask.astype(jnp.int32)
    s1, s2, cnt = _k_reduce(inputs, ids)
    table = pl.pallas_call(
        _k_table_body,
        out_shape=jax.ShapeDtypeStruct((NUM_SEGMENTS, 2 * D_FEAT), jnp.float32),
    )(s1, s2, cnt, bias.reshape(1, D_FEAT))
    return _k_apply(inputs, ids, table)
